# Initial kernel scaffold; baseline (speedup 1.0000x reference)
import jax
import jax.numpy as jnp
from jax.experimental import pallas as pl


def kernel(x, table):
    raise NotImplementedError("write your pallas kernel here")


# speedup vs baseline: 1.4826x; 1.4826x over previous
import functools

import jax
import jax.numpy as jnp
from jax import lax
from jax.experimental import pallas as pl
from jax.experimental.pallas import tpu as pltpu
from jax.experimental.pallas import tpu_sc as plsc

_D = 32
_NC = 2
_NS = 16
_NW = _NC * _NS
_RPD = 128
_GPG = 10
_GROUP = _GPG * _RPD


@functools.cache
def _make_emb(n_rows):
    per_w = n_rows // _NW
    steps = per_w // _RPD
    ngroups = steps // _GPG
    mesh = plsc.VectorSubcoreMesh(core_axis_name="c", subcore_axis_name="s")

    @functools.partial(
        pl.kernel,
        mesh=mesh,
        out_type=jax.ShapeDtypeStruct((n_rows, _D), jnp.float32),
        scratch_types=[
            pltpu.VMEM((steps, _RPD), jnp.int32),
            pltpu.VMEM((_GROUP, _D), jnp.float32),
            pltpu.SemaphoreType.DMA,
        ],
        compiler_params=pltpu.CompilerParams(use_tc_tiling_on_sc=False),
    )
    def emb(idx_hbm, table_hbm, out_hbm, idx_v, rows_v, gsem):
        wid = lax.axis_index("s") * _NC + lax.axis_index("c")
        base = wid * per_w
        pltpu.sync_copy(idx_hbm.at[wid], idx_v)

        def body(g, carry):
            handles = [
                pltpu.async_copy(
                    table_hbm.at[idx_v.at[g * _GPG + i]],
                    rows_v.at[pl.ds(i * _RPD, _RPD)],
                    gsem,
                )
                for i in range(_GPG)
            ]
            for h in handles:
                h.wait()
            pltpu.sync_copy(rows_v, out_hbm.at[pl.ds(base + g * _GROUP, _GROUP)])
            return carry

        lax.fori_loop(0, ngroups, body, 0)

    return emb


def kernel(x, table):
    b, l = x.shape
    n = b * l
    idx = x.reshape(_NW, n // (_NW * _RPD), _RPD)
    out = _make_emb(n)(idx, table)
    return out.reshape(b, l, _D)
